# baseline (speedup 1.0000x reference)
import functools
import math

import jax
import jax.numpy as jnp
from jax import lax
from jax.experimental import pallas as pl
from jax.experimental.pallas import tpu as pltpu
from jax.experimental.pallas import tpu_sc as plsc

_B = 16384
_D = 16
_EPS = 1e-5
_N_USERS = 1000000
_V = 2000000


def _sc_cross(tflat, idx_u, idx_i, fs_b, fb_b):
    info = plsc.get_sparse_core_info()
    nc, ns = info.num_cores, info.num_subcores
    nw = nc * ns
    bpw = _B // nw
    nv = bpw // _D
    mesh = plsc.VectorSubcoreMesh(core_axis_name="c", subcore_axis_name="s")

    @functools.partial(
        pl.kernel,
        mesh=mesh,
        compiler_params=pltpu.CompilerParams(use_tc_tiling_on_sc=False),
        out_type=jax.ShapeDtypeStruct((_D, _B), jnp.float32),
        scratch_types=[
            pltpu.VMEM((bpw,), jnp.int32),
            pltpu.VMEM((bpw,), jnp.int32),
            pltpu.VMEM((_D, bpw), jnp.int32),
            pltpu.VMEM((_D, bpw), jnp.int32),
            pltpu.VMEM((_D, bpw), jnp.float32),
            pltpu.VMEM((_D, bpw), jnp.float32),
            pltpu.VMEM((_D, bpw), jnp.float32),
            pltpu.VMEM((_D, _D), jnp.float32),
            pltpu.VMEM((_D, _D), jnp.float32),
            pltpu.SemaphoreType.DMA,
        ],
    )
    def k(t_hbm, iu_hbm, ii_hbm, fs_hbm, fb_hbm, cross_hbm,
          iu_v, ii_v, au_v, ai_v, cu_v, ci_v, cr_v, fs_v, fb_v, sem):
        wid = lax.axis_index("s") * nc + lax.axis_index("c")
        base = wid * bpw
        pltpu.sync_copy(iu_hbm.at[pl.ds(base, bpw)], iu_v)
        pltpu.sync_copy(ii_hbm.at[pl.ds(base, bpw)], ii_v)
        pltpu.sync_copy(fs_hbm, fs_v)
        pltpu.sync_copy(fb_hbm, fb_v)

        def addr_body(j, _):
            s = pl.ds(j * _D, _D)
            for iv, av in ((iu_v, au_v), (ii_v, ai_v)):
                r = iv[s]
                b = ((r >> 7) << 10) | (r & 127)
                for c in range(_D):
                    off = (c // 8) * 16000000 + (c % 8) * 128
                    av[c, s] = b + off
            return 0

        lax.fori_loop(0, nv, addr_body, 0, unroll=True)

        cps = []
        for c in range(_D):
            cps.append(pltpu.async_copy(t_hbm.at[au_v.at[c]], cu_v.at[c], sem))
            cps.append(pltpu.async_copy(t_hbm.at[ai_v.at[c]], ci_v.at[c], sem))
        for cp in cps:
            cp.wait()

        def cross_body(j, _):
            s = pl.ds(j * _D, _D)
            for c in range(_D):
                cr_v[c, s] = cu_v[c, s] * ci_v[c, s] * fs_v[c] + fb_v[c]
            return 0

        lax.fori_loop(0, nv, cross_body, 0, unroll=True)
        pltpu.sync_copy(cr_v, cross_hbm.at[:, pl.ds(base, bpw)])

    return k(tflat, idx_u, idx_i, fs_b, fb_b)


def _sc_lin(lin8, idx_u, idx_i):
    info = plsc.get_sparse_core_info()
    nc, ns = info.num_cores, info.num_subcores
    nw = nc * ns
    bpw = _B // nw
    mesh = plsc.VectorSubcoreMesh(core_axis_name="c", subcore_axis_name="s")

    @functools.partial(
        pl.kernel,
        mesh=mesh,
        compiler_params=pltpu.CompilerParams(use_tc_tiling_on_sc=False),
        out_type=[
            jax.ShapeDtypeStruct((_B, 8), jnp.float32),
            jax.ShapeDtypeStruct((_B, 8), jnp.float32),
        ],
        scratch_types=[
            pltpu.VMEM((bpw,), jnp.int32),
            pltpu.VMEM((bpw,), jnp.int32),
            pltpu.VMEM((bpw, 8), jnp.float32),
            pltpu.VMEM((bpw, 8), jnp.float32),
            pltpu.SemaphoreType.DMA,
        ],
    )
    def k(lin_hbm, iu_hbm, ii_hbm, lu_hbm, li_hbm,
          iu_v, ii_v, lu_v, li_v, sem):
        wid = lax.axis_index("s") * nc + lax.axis_index("c")
        base = wid * bpw
        pltpu.sync_copy(iu_hbm.at[pl.ds(base, bpw)], iu_v)
        pltpu.sync_copy(ii_hbm.at[pl.ds(base, bpw)], ii_v)
        cp1 = pltpu.async_copy(lin_hbm.at[iu_v], lu_v, sem)
        cp2 = pltpu.async_copy(lin_hbm.at[ii_v], li_v, sem)
        cp1.wait()
        cp2.wait()
        pltpu.sync_copy(lu_v, lu_hbm.at[pl.ds(base, bpw), :])
        pltpu.sync_copy(li_v, li_hbm.at[pl.ds(base, bpw), :])

    return k(lin8, idx_u, idx_i)


def _tc_mlp_body(cr_ref, ls_ref, w1_ref, a1_ref, c1_ref, w2_ref, a2_ref,
                 c2_ref, w3_ref, out_ref):
    h = lax.dot_general(w1_ref[...], cr_ref[...], (((1,), (0,)), ((), ())),
                        preferred_element_type=jnp.float32)
    h = jnp.maximum(h * a1_ref[...] + c1_ref[...], 0.0)
    h = lax.dot_general(w2_ref[...], h, (((1,), (0,)), ((), ())),
                        preferred_element_type=jnp.float32)
    h = jnp.maximum(h * a2_ref[...] + c2_ref[...], 0.0)
    z = jnp.sum(h * w3_ref[...], axis=0, keepdims=True) + ls_ref[...]
    out_ref[...] = jax.nn.sigmoid(z)


def kernel(users_feat, items_feat, emb_table, lin_table, lin_bias, fm_gamma,
           fm_beta, W1, b1, g1, bt1, W2, b2, g2, bt2, W3, b3):
    k = 1.0 / math.sqrt(1.0 + _EPS)
    idx_u = users_feat.astype(jnp.int32)
    idx_i = items_feat.astype(jnp.int32) + jnp.int32(_N_USERS)
    tflat = (emb_table.T.reshape(2, 8, _V // 128, 128)
             .transpose(0, 2, 1, 3).reshape(_V * _D))
    fs_b = jnp.broadcast_to((fm_gamma * k)[:, None], (_D, _D))
    fb_b = jnp.broadcast_to(fm_beta[:, None], (_D, _D))
    lin8 = jnp.pad(lin_table, ((0, 0), (0, 7)))

    crossT = _sc_cross(tflat, idx_u, idx_i, fs_b, fb_b)
    lu8, li8 = _sc_lin(lin8, idx_u, idx_i)
    ls = ((lu8 + li8).sum(axis=1) + (lin_bias[0] + b3[0])).reshape(1, _B)

    a1 = (g1 * k)[:, None]
    c1 = (b1 * g1 * k + bt1)[:, None]
    a2 = (g2 * k)[:, None]
    c2 = (b2 * g2 * k + bt2)[:, None]
    w3 = W3.reshape(-1)[:, None]

    out = pl.pallas_call(
        _tc_mlp_body,
        out_shape=jax.ShapeDtypeStruct((1, _B), jnp.float32),
    )(crossT, ls, W1, a1, c1, W2, a2, c2, w3)
    return out.reshape(_B)


# speedup vs baseline: 26.7071x; 26.7071x over previous
import functools
import math

import jax
import jax.numpy as jnp
from jax import lax
from jax.experimental import pallas as pl
from jax.experimental.pallas import tpu as pltpu
from jax.experimental.pallas import tpu_sc as plsc

_B = 16384
_D = 16
_EPS = 1e-5
_N_USERS = 1000000
_V = 2000000


def _sc_cross(tflat, idx_u, idx_i, fs_b, fb_b):
    info = plsc.get_sparse_core_info()
    nc, ns = info.num_cores, info.num_subcores
    nw = nc * ns
    bpw = _B // nw
    nv = bpw // _D
    mesh = plsc.VectorSubcoreMesh(core_axis_name="c", subcore_axis_name="s")

    @functools.partial(
        pl.kernel,
        mesh=mesh,
        compiler_params=pltpu.CompilerParams(use_tc_tiling_on_sc=False),
        out_type=jax.ShapeDtypeStruct((_D, _B), jnp.float32),
        scratch_types=[
            pltpu.VMEM((bpw,), jnp.int32),
            pltpu.VMEM((bpw,), jnp.int32),
            pltpu.VMEM((_D, bpw), jnp.int32),
            pltpu.VMEM((_D, bpw), jnp.int32),
            pltpu.VMEM((_D, bpw), jnp.float32),
            pltpu.VMEM((_D, bpw), jnp.float32),
            pltpu.VMEM((_D, bpw), jnp.float32),
            pltpu.VMEM((_D, _D), jnp.float32),
            pltpu.VMEM((_D, _D), jnp.float32),
            pltpu.SemaphoreType.DMA,
        ],
    )
    def k(t_hbm, iu_hbm, ii_hbm, fs_hbm, fb_hbm, cross_hbm,
          iu_v, ii_v, au_v, ai_v, cu_v, ci_v, cr_v, fs_v, fb_v, sem):
        wid = lax.axis_index("s") * nc + lax.axis_index("c")
        base = wid * bpw
        pltpu.sync_copy(iu_hbm.at[pl.ds(base, bpw)], iu_v)
        pltpu.sync_copy(ii_hbm.at[pl.ds(base, bpw)], ii_v)
        pltpu.sync_copy(fs_hbm, fs_v)
        pltpu.sync_copy(fb_hbm, fb_v)

        def addr_body(j, _):
            s = pl.ds(j * _D, _D)
            for iv, av in ((iu_v, au_v), (ii_v, ai_v)):
                r = iv[s]
                b = ((r >> 7) << 10) | (r & 127)
                for c in range(_D):
                    off = (c // 8) * 16000000 + (c % 8) * 128
                    av[c, s] = b + off
            return 0

        lax.fori_loop(0, nv, addr_body, 0, unroll=True)

        cps = []
        for c in range(_D):
            cps.append(pltpu.async_copy(t_hbm.at[au_v.at[c]], cu_v.at[c], sem))
            cps.append(pltpu.async_copy(t_hbm.at[ai_v.at[c]], ci_v.at[c], sem))
        for cp in cps:
            cp.wait()

        def cross_body(j, _):
            s = pl.ds(j * _D, _D)
            for c in range(_D):
                cr_v[c, s] = cu_v[c, s] * ci_v[c, s] * fs_v[c] + fb_v[c]
            return 0

        lax.fori_loop(0, nv, cross_body, 0, unroll=True)
        pltpu.sync_copy(cr_v, cross_hbm.at[:, pl.ds(base, bpw)])

    return k(tflat, idx_u, idx_i, fs_b, fb_b)


def _sc_lin(lin_flat, idx_u, idx_i):
    info = plsc.get_sparse_core_info()
    nc, ns = info.num_cores, info.num_subcores
    nw = nc * ns
    bpw = _B // nw
    nv = bpw // _D
    mesh = plsc.VectorSubcoreMesh(core_axis_name="c", subcore_axis_name="s")

    @functools.partial(
        pl.kernel,
        mesh=mesh,
        compiler_params=pltpu.CompilerParams(use_tc_tiling_on_sc=False),
        out_type=jax.ShapeDtypeStruct((_B,), jnp.float32),
        scratch_types=[
            pltpu.VMEM((bpw,), jnp.int32),
            pltpu.VMEM((bpw,), jnp.int32),
            pltpu.VMEM((bpw,), jnp.float32),
            pltpu.VMEM((bpw,), jnp.float32),
            pltpu.VMEM((bpw,), jnp.float32),
            pltpu.SemaphoreType.DMA,
        ],
    )
    def k(lin_hbm, iu_hbm, ii_hbm, lsum_hbm,
          iu_v, ii_v, lu_v, li_v, ls_v, sem):
        wid = lax.axis_index("s") * nc + lax.axis_index("c")
        base = wid * bpw
        pltpu.sync_copy(iu_hbm.at[pl.ds(base, bpw)], iu_v)
        pltpu.sync_copy(ii_hbm.at[pl.ds(base, bpw)], ii_v)
        cp1 = pltpu.async_copy(lin_hbm.at[iu_v], lu_v, sem)
        cp2 = pltpu.async_copy(lin_hbm.at[ii_v], li_v, sem)
        cp1.wait()
        cp2.wait()

        def lin_body(j, _):
            s = pl.ds(j * _D, _D)
            ls_v[s] = lu_v[s] + li_v[s]
            return 0

        lax.fori_loop(0, nv, lin_body, 0, unroll=True)
        pltpu.sync_copy(ls_v, lsum_hbm.at[pl.ds(base, bpw)])

    return k(lin_flat, idx_u, idx_i)


def _tc_mlp_body(cr_ref, ls_ref, bias_ref, w1_ref, a1_ref, c1_ref,
                 w2_ref, a2_ref, c2_ref, w3_ref, out_ref):
    h = lax.dot_general(cr_ref[...], w1_ref[...], (((0,), (1,)), ((), ())),
                        preferred_element_type=jnp.float32)
    h = jnp.maximum(h * a1_ref[...] + c1_ref[...], 0.0)
    h = lax.dot_general(h, w2_ref[...], (((1,), (1,)), ((), ())),
                        preferred_element_type=jnp.float32)
    h = jnp.maximum(h * a2_ref[...] + c2_ref[...], 0.0)
    z = jnp.sum(h * w3_ref[...], axis=1) + ls_ref[...] + bias_ref[...]
    out_ref[...] = jax.nn.sigmoid(z)


def kernel(users_feat, items_feat, emb_table, lin_table, lin_bias, fm_gamma,
           fm_beta, W1, b1, g1, bt1, W2, b2, g2, bt2, W3, b3):
    k = 1.0 / math.sqrt(1.0 + _EPS)
    idx_u = users_feat.astype(jnp.int32)
    idx_i = items_feat.astype(jnp.int32) + jnp.int32(_N_USERS)
    tflat = (emb_table.T.reshape(2, 8, _V // 128, 128)
             .transpose(0, 2, 1, 3).reshape(_V * _D))
    fs_b = jnp.broadcast_to((fm_gamma * k)[:, None], (_D, _D))
    fb_b = jnp.broadcast_to(fm_beta[:, None], (_D, _D))
    lin_flat = lin_table.reshape(_V)

    crossT = _sc_cross(tflat, idx_u, idx_i, fs_b, fb_b)
    lsum = _sc_lin(lin_flat, idx_u, idx_i)
    bias1 = lin_bias + b3

    a1 = (g1 * k)[None, :]
    c1 = (b1 * g1 * k + bt1)[None, :]
    a2 = (g2 * k)[None, :]
    c2 = (b2 * g2 * k + bt2)[None, :]
    w3 = W3.reshape(-1)[None, :]

    out = pl.pallas_call(
        _tc_mlp_body,
        out_shape=jax.ShapeDtypeStruct((_B,), jnp.float32),
    )(crossT, lsum, bias1, W1, a1, c1, W2, a2, c2, w3)
    return out


# speedup vs baseline: 29.4950x; 1.1044x over previous
import functools
import math

import jax
import jax.numpy as jnp
from jax import lax
from jax.experimental import pallas as pl
from jax.experimental.pallas import tpu as pltpu
from jax.experimental.pallas import tpu_sc as plsc

_B = 16384
_D = 16
_EPS = 1e-5
_N_USERS = 1000000
_V = 2000000


def _sc_cross(tflat, idx_u, idx_i, fs_b, fb_b):
    info = plsc.get_sparse_core_info()
    nc, ns = info.num_cores, info.num_subcores
    nw = nc * ns
    bpw = _B // nw
    nv = bpw // _D
    mesh = plsc.VectorSubcoreMesh(core_axis_name="c", subcore_axis_name="s")

    @functools.partial(
        pl.kernel,
        mesh=mesh,
        compiler_params=pltpu.CompilerParams(use_tc_tiling_on_sc=False),
        out_type=jax.ShapeDtypeStruct((_D, _B), jnp.float32),
        scratch_types=[
            pltpu.VMEM((bpw,), jnp.int32),
            pltpu.VMEM((bpw,), jnp.int32),
            pltpu.VMEM((_D, bpw), jnp.int32),
            pltpu.VMEM((_D, bpw), jnp.int32),
            pltpu.VMEM((_D, bpw), jnp.float32),
            pltpu.VMEM((_D, bpw), jnp.float32),
            pltpu.VMEM((_D, bpw), jnp.float32),
            pltpu.VMEM((_D, _D), jnp.float32),
            pltpu.VMEM((_D, _D), jnp.float32),
            pltpu.SemaphoreType.DMA,
        ],
    )
    def k(t_hbm, iu_hbm, ii_hbm, fs_hbm, fb_hbm, cross_hbm,
          iu_v, ii_v, au_v, ai_v, cu_v, ci_v, cr_v, fs_v, fb_v, sem):
        wid = lax.axis_index("s") * nc + lax.axis_index("c")
        base = wid * bpw
        pltpu.sync_copy(iu_hbm.at[pl.ds(base, bpw)], iu_v)
        pltpu.sync_copy(ii_hbm.at[pl.ds(base, bpw)], ii_v)
        pltpu.sync_copy(fs_hbm, fs_v)
        pltpu.sync_copy(fb_hbm, fb_v)

        def addr_body(j, _):
            s = pl.ds(j * _D, _D)
            for iv, av in ((iu_v, au_v), (ii_v, ai_v)):
                r = iv[s]
                b = ((r >> 7) << 10) | (r & 127)
                for c in range(_D):
                    off = (c // 8) * 16000000 + (c % 8) * 128
                    av[c, s] = b + off
            return 0

        lax.fori_loop(0, nv, addr_body, 0, unroll=True)

        cps = []
        for c in range(_D):
            cps.append(pltpu.async_copy(t_hbm.at[au_v.at[c]], cu_v.at[c], sem))
            cps.append(pltpu.async_copy(t_hbm.at[ai_v.at[c]], ci_v.at[c], sem))
        for cp in cps:
            cp.wait()

        def cross_body(j, _):
            s = pl.ds(j * _D, _D)
            for c in range(_D):
                cr_v[c, s] = cu_v[c, s] * ci_v[c, s] * fs_v[c] + fb_v[c]
            return 0

        lax.fori_loop(0, nv, cross_body, 0, unroll=True)
        pltpu.sync_copy(cr_v, cross_hbm.at[:, pl.ds(base, bpw)])

    return k(tflat, idx_u, idx_i, fs_b, fb_b)


def _sc_lin(lin_flat, idx_u, idx_i):
    info = plsc.get_sparse_core_info()
    nc, ns = info.num_cores, info.num_subcores
    nw = nc * ns
    bpw = _B // nw
    nv = bpw // _D
    mesh = plsc.VectorSubcoreMesh(core_axis_name="c", subcore_axis_name="s")

    @functools.partial(
        pl.kernel,
        mesh=mesh,
        compiler_params=pltpu.CompilerParams(use_tc_tiling_on_sc=False),
        out_type=jax.ShapeDtypeStruct((_B,), jnp.float32),
        scratch_types=[
            pltpu.VMEM((bpw,), jnp.int32),
            pltpu.VMEM((bpw,), jnp.int32),
            pltpu.VMEM((bpw,), jnp.float32),
            pltpu.VMEM((bpw,), jnp.float32),
            pltpu.VMEM((bpw,), jnp.float32),
            pltpu.SemaphoreType.DMA,
        ],
    )
    def k(lin_hbm, iu_hbm, ii_hbm, lsum_hbm,
          iu_v, ii_v, lu_v, li_v, ls_v, sem):
        wid = lax.axis_index("s") * nc + lax.axis_index("c")
        base = wid * bpw
        pltpu.sync_copy(iu_hbm.at[pl.ds(base, bpw)], iu_v)
        pltpu.sync_copy(ii_hbm.at[pl.ds(base, bpw)], ii_v)
        cp1 = pltpu.async_copy(lin_hbm.at[iu_v], lu_v, sem)
        cp2 = pltpu.async_copy(lin_hbm.at[ii_v], li_v, sem)
        cp1.wait()
        cp2.wait()

        def lin_body(j, _):
            s = pl.ds(j * _D, _D)
            ls_v[s] = lu_v[s] + li_v[s]
            return 0

        lax.fori_loop(0, nv, lin_body, 0, unroll=True)
        pltpu.sync_copy(ls_v, lsum_hbm.at[pl.ds(base, bpw)])

    return k(lin_flat, idx_u, idx_i)


def _tc_mlp_body(cr_ref, ls_ref, w1_ref, a1_ref, c1_ref, w2_ref, a2_ref,
                 c2_ref, w3_ref, out_ref):
    h = lax.dot_general(w1_ref[...], cr_ref[...], (((1,), (0,)), ((), ())),
                        preferred_element_type=jnp.float32)
    h = jnp.maximum(h * a1_ref[...] + c1_ref[...], 0.0)
    h = lax.dot_general(w2_ref[...], h, (((1,), (0,)), ((), ())),
                        preferred_element_type=jnp.float32)
    h = jnp.maximum(h * a2_ref[...] + c2_ref[...], 0.0)
    z = jnp.sum(h * w3_ref[...], axis=0, keepdims=True) + ls_ref[...]
    out_ref[...] = jax.nn.sigmoid(z)


def kernel(users_feat, items_feat, emb_table, lin_table, lin_bias, fm_gamma,
           fm_beta, W1, b1, g1, bt1, W2, b2, g2, bt2, W3, b3):
    k = 1.0 / math.sqrt(1.0 + _EPS)
    idx_u = users_feat.astype(jnp.int32)
    idx_i = items_feat.astype(jnp.int32) + jnp.int32(_N_USERS)
    tflat = (emb_table.T.reshape(2, 8, _V // 128, 128)
             .transpose(0, 2, 1, 3).reshape(_V * _D))
    fs_b = jnp.broadcast_to((fm_gamma * k)[:, None], (_D, _D))
    fb_b = jnp.broadcast_to(fm_beta[:, None], (_D, _D))
    lin_flat = lin_table.reshape(_V)

    crossT = _sc_cross(tflat, idx_u, idx_i, fs_b, fb_b)
    lsum = _sc_lin(lin_flat, idx_u, idx_i)
    ls = (lsum + (lin_bias[0] + b3[0])).reshape(1, _B)

    a1 = (g1 * k)[:, None]
    c1 = (b1 * g1 * k + bt1)[:, None]
    a2 = (g2 * k)[:, None]
    c2 = (b2 * g2 * k + bt2)[:, None]
    w3 = W3.reshape(-1)[:, None]

    out = pl.pallas_call(
        _tc_mlp_body,
        out_shape=jax.ShapeDtypeStruct((1, _B), jnp.float32),
    )(crossT, ls, W1, a1, c1, W2, a2, c2, w3)
    return out.reshape(_B)


# speedup vs baseline: 29.5913x; 1.0033x over previous
import functools
import math

import jax
import jax.numpy as jnp
from jax import lax
from jax.experimental import pallas as pl
from jax.experimental.pallas import tpu as pltpu
from jax.experimental.pallas import tpu_sc as plsc

_B = 16384
_D = 16
_EPS = 1e-5
_N_USERS = 1000000
_V = 2000000


def _sc_cross(tflat, idx_u, idx_i, fs_b, fb_b):
    info = plsc.get_sparse_core_info()
    nc, ns = info.num_cores, info.num_subcores
    nw = nc * ns
    bpw = _B // nw
    nv = bpw // _D
    mesh = plsc.VectorSubcoreMesh(core_axis_name="c", subcore_axis_name="s")

    @functools.partial(
        pl.kernel,
        mesh=mesh,
        compiler_params=pltpu.CompilerParams(use_tc_tiling_on_sc=False),
        out_type=jax.ShapeDtypeStruct((_D, _B), jnp.float32),
        scratch_types=[
            pltpu.VMEM((bpw,), jnp.int32),
            pltpu.VMEM((bpw,), jnp.int32),
            pltpu.VMEM((_D, bpw), jnp.int32),
            pltpu.VMEM((_D, bpw), jnp.int32),
            pltpu.VMEM((_D, bpw), jnp.float32),
            pltpu.VMEM((_D, bpw), jnp.float32),
            pltpu.VMEM((_D, bpw), jnp.float32),
            pltpu.VMEM((_D, _D), jnp.float32),
            pltpu.VMEM((_D, _D), jnp.float32),
            pltpu.SemaphoreType.DMA,
        ],
    )
    def k(t_hbm, iu_hbm, ii_hbm, fs_hbm, fb_hbm, cross_hbm,
          iu_v, ii_v, au_v, ai_v, cu_v, ci_v, cr_v, fs_v, fb_v, sem):
        wid = lax.axis_index("s") * nc + lax.axis_index("c")
        base = wid * bpw
        pltpu.sync_copy(iu_hbm.at[pl.ds(base, bpw)], iu_v)
        pltpu.sync_copy(ii_hbm.at[pl.ds(base, bpw)], ii_v)
        pltpu.sync_copy(fs_hbm, fs_v)
        pltpu.sync_copy(fb_hbm, fb_v)

        def addr_body(j, _):
            s = pl.ds(j * _D, _D)
            for iv, av in ((iu_v, au_v), (ii_v, ai_v)):
                r = iv[s]
                b = ((r >> 7) << 10) | (r & 127)
                for c in range(_D):
                    off = (c // 8) * 16000000 + (c % 8) * 128
                    av[c, s] = b + off
            return 0

        lax.fori_loop(0, nv, addr_body, 0, unroll=True)

        cps = []
        for c in range(_D):
            cps.append(pltpu.async_copy(t_hbm.at[au_v.at[c]], cu_v.at[c], sem))
            cps.append(pltpu.async_copy(t_hbm.at[ai_v.at[c]], ci_v.at[c], sem))
        for cp in cps:
            cp.wait()

        def cross_body(j, _):
            s = pl.ds(j * _D, _D)
            for c in range(_D):
                cr_v[c, s] = cu_v[c, s] * ci_v[c, s] * fs_v[c] + fb_v[c]
            return 0

        lax.fori_loop(0, nv, cross_body, 0, unroll=True)
        pltpu.sync_copy(cr_v, cross_hbm.at[:, pl.ds(base, bpw)])

    return k(tflat, idx_u, idx_i, fs_b, fb_b)


def _sc_lin(lin_flat, idx_u, idx_i, bias_b):
    info = plsc.get_sparse_core_info()
    nc, ns = info.num_cores, info.num_subcores
    nw = nc * ns
    bpw = _B // nw
    nv = bpw // _D
    mesh = plsc.VectorSubcoreMesh(core_axis_name="c", subcore_axis_name="s")

    @functools.partial(
        pl.kernel,
        mesh=mesh,
        compiler_params=pltpu.CompilerParams(use_tc_tiling_on_sc=False),
        out_type=jax.ShapeDtypeStruct((1, _B), jnp.float32),
        scratch_types=[
            pltpu.VMEM((bpw,), jnp.int32),
            pltpu.VMEM((bpw,), jnp.int32),
            pltpu.VMEM((bpw,), jnp.float32),
            pltpu.VMEM((bpw,), jnp.float32),
            pltpu.VMEM((bpw,), jnp.float32),
            pltpu.VMEM((_D,), jnp.float32),
            pltpu.SemaphoreType.DMA,
        ],
    )
    def k(lin_hbm, iu_hbm, ii_hbm, bias_hbm, lsum_hbm,
          iu_v, ii_v, lu_v, li_v, ls_v, bias_v, sem):
        wid = lax.axis_index("s") * nc + lax.axis_index("c")
        base = wid * bpw
        pltpu.sync_copy(iu_hbm.at[pl.ds(base, bpw)], iu_v)
        pltpu.sync_copy(ii_hbm.at[pl.ds(base, bpw)], ii_v)
        pltpu.sync_copy(bias_hbm, bias_v)
        cp1 = pltpu.async_copy(lin_hbm.at[iu_v], lu_v, sem)
        cp2 = pltpu.async_copy(lin_hbm.at[ii_v], li_v, sem)
        cp1.wait()
        cp2.wait()
        bias = bias_v[...]

        def lin_body(j, _):
            s = pl.ds(j * _D, _D)
            ls_v[s] = lu_v[s] + li_v[s] + bias
            return 0

        lax.fori_loop(0, nv, lin_body, 0, unroll=True)
        pltpu.sync_copy(ls_v, lsum_hbm.at[0, pl.ds(base, bpw)])

    return k(lin_flat, idx_u, idx_i, bias_b)


def _tc_mlp_body(cr_ref, ls_ref, w1_ref, a1_ref, c1_ref, w2_ref, a2_ref,
                 c2_ref, w3_ref, out_ref):
    h = lax.dot_general(w1_ref[...], cr_ref[...], (((1,), (0,)), ((), ())),
                        preferred_element_type=jnp.float32)
    h = jnp.maximum(h * a1_ref[...] + c1_ref[...], 0.0)
    h = lax.dot_general(w2_ref[...], h, (((1,), (0,)), ((), ())),
                        preferred_element_type=jnp.float32)
    h = jnp.maximum(h * a2_ref[...] + c2_ref[...], 0.0)
    z = jnp.sum(h * w3_ref[...], axis=0, keepdims=True) + ls_ref[...]
    out_ref[...] = jax.nn.sigmoid(z)


def kernel(users_feat, items_feat, emb_table, lin_table, lin_bias, fm_gamma,
           fm_beta, W1, b1, g1, bt1, W2, b2, g2, bt2, W3, b3):
    k = 1.0 / math.sqrt(1.0 + _EPS)
    idx_u = users_feat.astype(jnp.int32)
    idx_i = items_feat.astype(jnp.int32) + jnp.int32(_N_USERS)
    tflat = (emb_table.T.reshape(2, 8, _V // 128, 128)
             .transpose(0, 2, 1, 3).reshape(_V * _D))
    fs_b = jnp.broadcast_to((fm_gamma * k)[:, None], (_D, _D))
    fb_b = jnp.broadcast_to(fm_beta[:, None], (_D, _D))
    lin_flat = lin_table.reshape(_V)

    bias_b = jnp.broadcast_to(lin_bias[0] + b3[0], (_D,))

    crossT = _sc_cross(tflat, idx_u, idx_i, fs_b, fb_b)
    ls = _sc_lin(lin_flat, idx_u, idx_i, bias_b)

    a1 = (g1 * k)[:, None]
    c1 = (b1 * g1 * k + bt1)[:, None]
    a2 = (g2 * k)[:, None]
    c2 = (b2 * g2 * k + bt2)[:, None]
    w3 = W3.reshape(-1)[:, None]

    out = pl.pallas_call(
        _tc_mlp_body,
        out_shape=jax.ShapeDtypeStruct((1, _B), jnp.float32),
    )(crossT, ls, W1, a1, c1, W2, a2, c2, w3)
    return out.reshape(_B)
